# Initial kernel scaffold; baseline (speedup 1.0000x reference)
import jax
import jax.numpy as jnp
from jax.experimental import pallas as pl


def kernel(x, edge_index, edge_attr, batch, W_in, b_in, W_edge, W1, b1, W2, b2, Wp, bp):
    raise NotImplementedError("write your pallas kernel here")


# speedup vs baseline: 3.7715x; 3.7715x over previous
import functools

import jax
import jax.numpy as jnp
from jax import lax
from jax.experimental import pallas as pl
from jax.experimental.pallas import tpu as pltpu
from jax.experimental.pallas import tpu_sc as plsc

NC = 2
NS = 16
NW = NC * NS
CH = 80
WH = 160


def _mesh():
    return plsc.VectorSubcoreMesh(
        core_axis_name="c", subcore_axis_name="s",
        num_cores=NC, num_subcores=NS)


def _zero_vmem(ref, rows, width):
    zv = jnp.zeros((16,), jnp.float32)

    def zi(i, _):
        def zj(j, _):
            ref[i, pl.ds(j * 16, 16)] = zv
            return 0
        return lax.fori_loop(0, width // 16, zj, 0)

    lax.fori_loop(0, rows, zi, 0)


def _sc_edge_sum(N, E, DE):
    EW = E // NW
    NCH = EW // CH
    RPT = N // NS

    @functools.partial(
        pl.kernel,
        out_type=jax.ShapeDtypeStruct((NC, N, DE), jnp.float32),
        mesh=_mesh(),
        compiler_params=pltpu.CompilerParams(use_tc_tiling_on_sc=False),
        scratch_types=[
            pltpu.VMEM((CH, DE), jnp.float32),
            pltpu.VMEM((CH,), jnp.int32),
            pltpu.VMEM((RPT, DE), jnp.float32),
            pltpu.VMEM_SHARED((N, DE), jnp.float32),
        ],
    )
    def k(ea_hbm, dst_hbm, out, rows, dstb, zb, acc):
        c = lax.axis_index("c")
        s = lax.axis_index("s")
        wid = c * NS + s
        _zero_vmem(zb, RPT, DE)
        pltpu.sync_copy(zb, acc.at[pl.ds(s * RPT, RPT)])
        plsc.subcore_barrier()

        def body(i, _):
            base = wid * EW + i * CH
            pltpu.sync_copy(dst_hbm.at[pl.ds(base, CH)], dstb)
            pltpu.sync_copy(ea_hbm.at[pl.ds(base, CH)], rows)
            pltpu.sync_copy(rows, acc.at[dstb], add=True)
            return 0

        lax.fori_loop(0, NCH, body, 0)
        plsc.subcore_barrier()
        pltpu.sync_copy(acc.at[pl.ds(s * RPT, RPT)],
                        out.at[c, pl.ds(s * RPT, RPT)])

    return k


def _sc_layer(N, E):
    EW = E // NW
    NCH = EW // CH
    RPT = N // NS
    RZ = 25
    RO = 125

    @functools.partial(
        pl.kernel,
        out_type=jax.ShapeDtypeStruct((2, NC, N, WH), jnp.float32),
        mesh=_mesh(),
        compiler_params=pltpu.CompilerParams(use_tc_tiling_on_sc=False),
        scratch_types=[
            pltpu.VMEM((CH, WH), jnp.float32),
            pltpu.VMEM((CH,), jnp.int32),
            pltpu.VMEM((CH,), jnp.int32),
            pltpu.VMEM((RZ, WH), jnp.float32),
            pltpu.VMEM_SHARED((N, WH), jnp.float32),
            pltpu.SemaphoreType.DMA,
        ],
    )
    def k(hA, hB, src_hbm, dst_hbm, out, rows, srcb, dstb, zb, acc, sem):
        c = lax.axis_index("c")
        s = lax.axis_index("s")
        wid = c * NS + s
        _zero_vmem(zb, RZ, WH)
        for p in range(2):
            h = hA if p == 0 else hB
            for j in range(RPT // RZ):
                pltpu.sync_copy(zb, acc.at[pl.ds(s * RPT + j * RZ, RZ)])
            plsc.subcore_barrier()

            def body(i, _):
                base = wid * EW + i * CH
                pltpu.sync_copy(src_hbm.at[pl.ds(base, CH)], srcb)
                pltpu.sync_copy(dst_hbm.at[pl.ds(base, CH)], dstb)
                pltpu.async_copy(h.at[srcb], rows, sem).wait()
                pltpu.sync_copy(rows, acc.at[dstb], add=True)
                return 0

            lax.fori_loop(0, NCH, body, 0)
            plsc.subcore_barrier()
            for j in range(RPT // RO):
                r = s * RPT + j * RO
                pltpu.sync_copy(acc.at[pl.ds(r, RO)],
                                out.at[p, c, pl.ds(r, RO)])
            if p == 0:
                plsc.subcore_barrier()

    return k


def _tc_pre(N, R, DF, EMB, DE):
    def body(x_ref, win_ref, bin_ref, ap_ref, hA_ref, hB_ref, asum_ref):
        h = jnp.dot(x_ref[...], win_ref[...],
                    preferred_element_type=jnp.float32) + bin_ref[...]
        hA_ref[...] = h[:, :WH]
        hB_ref[...] = jnp.concatenate(
            [h[:, WH:], jnp.zeros((R, 2 * WH - EMB), jnp.float32)], axis=1)
        ap = ap_ref[...]
        asum_ref[...] = ap[0] + ap[1]

    return pl.pallas_call(
        body,
        grid=(N // R,),
        in_specs=[
            pl.BlockSpec((R, DF), lambda i: (i, 0)),
            pl.BlockSpec((DF, EMB), lambda i: (0, 0)),
            pl.BlockSpec((1, EMB), lambda i: (0, 0)),
            pl.BlockSpec((NC, R, DE), lambda i: (0, i, 0)),
        ],
        out_specs=[
            pl.BlockSpec((R, WH), lambda i: (i, 0)),
            pl.BlockSpec((R, WH), lambda i: (i, 0)),
            pl.BlockSpec((R, DE), lambda i: (i, 0)),
        ],
        out_shape=[
            jax.ShapeDtypeStruct((N, WH), jnp.float32),
            jax.ShapeDtypeStruct((N, WH), jnp.float32),
            jax.ShapeDtypeStruct((N, DE), jnp.float32),
        ],
    )


def _tc_mlp(N, R, EMB, DE, last):
    def body(aggp_ref, asum_ref, we_ref, w1_ref, b1_ref, w2_ref, b2_ref,
             hA_ref, hB_ref):
        ap = aggp_ref[...]
        agg_a = ap[0, 0] + ap[0, 1]
        agg_b = ap[1, 0] + ap[1, 1]
        agg = jnp.concatenate([agg_a, agg_b[:, :EMB - WH]], axis=1)
        agg = agg + jnp.dot(asum_ref[...], we_ref[...],
                            preferred_element_type=jnp.float32)
        y = jnp.maximum(jnp.dot(agg, w1_ref[...],
                                preferred_element_type=jnp.float32)
                        + b1_ref[...], 0.0)
        z = jnp.dot(y, w2_ref[...],
                    preferred_element_type=jnp.float32) + b2_ref[...]
        h = z if last else jnp.maximum(z, 0.0)
        hA_ref[...] = h[:, :WH]
        hB_ref[...] = jnp.concatenate(
            [h[:, WH:], jnp.zeros((R, 2 * WH - EMB), jnp.float32)], axis=1)

    return pl.pallas_call(
        body,
        grid=(N // R,),
        in_specs=[
            pl.BlockSpec((2, NC, R, WH), lambda i: (0, 0, i, 0)),
            pl.BlockSpec((R, DE), lambda i: (i, 0)),
            pl.BlockSpec((DE, EMB), lambda i: (0, 0)),
            pl.BlockSpec((EMB, 2 * EMB), lambda i: (0, 0)),
            pl.BlockSpec((1, 2 * EMB), lambda i: (0, 0)),
            pl.BlockSpec((2 * EMB, EMB), lambda i: (0, 0)),
            pl.BlockSpec((1, EMB), lambda i: (0, 0)),
        ],
        out_specs=[
            pl.BlockSpec((R, WH), lambda i: (i, 0)),
            pl.BlockSpec((R, WH), lambda i: (i, 0)),
        ],
        out_shape=[
            jax.ShapeDtypeStruct((N, WH), jnp.float32),
            jax.ShapeDtypeStruct((N, WH), jnp.float32),
        ],
    )


def _tc_final(N, R, EMB, PROJ):
    def body(hA_ref, hB_ref, wp_ref, bp_ref, out_ref):
        h = jnp.concatenate([hA_ref[...], hB_ref[...][:, :EMB - WH]], axis=1)
        o = jnp.dot(h, wp_ref[...],
                    preferred_element_type=jnp.float32) + bp_ref[...]
        n = jnp.maximum(
            jnp.sqrt(jnp.sum(o * o, axis=1, keepdims=True)), 1e-12)
        out_ref[...] = o / n

    return pl.pallas_call(
        body,
        grid=(N // R,),
        in_specs=[
            pl.BlockSpec((R, WH), lambda i: (i, 0)),
            pl.BlockSpec((R, WH), lambda i: (i, 0)),
            pl.BlockSpec((EMB, PROJ), lambda i: (0, 0)),
            pl.BlockSpec((1, PROJ), lambda i: (0, 0)),
        ],
        out_specs=pl.BlockSpec((R, PROJ), lambda i: (i, 0)),
        out_shape=jax.ShapeDtypeStruct((N, PROJ), jnp.float32),
    )


def kernel(x, edge_index, edge_attr, batch, W_in, b_in, W_edge, W1, b1,
           W2, b2, Wp, bp):
    N, DF = x.shape
    E = edge_index.shape[1]
    DE = edge_attr.shape[1]
    L, _, EMB = W_edge.shape
    PROJ = Wp.shape[1]
    R = 1000
    assert E % (NW * CH) == 0 and N % (NS * 125) == 0 and N % R == 0

    src = edge_index[0]
    dst = edge_index[1]

    ap = _sc_edge_sum(N, E, DE)(edge_attr, dst)
    hA, hB, asum = _tc_pre(N, R, DF, EMB, DE)(
        x, W_in, b_in.reshape(1, EMB), ap)

    layer_sc = _sc_layer(N, E)
    for l in range(L):
        parts = layer_sc(hA, hB, src, dst)
        hA, hB = _tc_mlp(N, R, EMB, DE, last=(l == L - 1))(
            parts, asum, W_edge[l], W1[l], b1[l].reshape(1, 2 * EMB),
            W2[l], b2[l].reshape(1, EMB))

    return _tc_final(N, R, EMB, PROJ)(hA, hB, Wp, bp.reshape(1, PROJ))


# speedup vs baseline: 6.9966x; 1.8551x over previous
import functools

import jax
import jax.numpy as jnp
from jax import lax
from jax.experimental import pallas as pl
from jax.experimental.pallas import tpu as pltpu
from jax.experimental.pallas import tpu_sc as plsc

NC = 2
NS = 16
NW = NC * NS
CH = 80
WH = 160


def _mesh():
    return plsc.VectorSubcoreMesh(
        core_axis_name="c", subcore_axis_name="s",
        num_cores=NC, num_subcores=NS)


def _zero_vmem(ref, rows, width):
    zv = jnp.zeros((16,), jnp.float32)

    def zi(i, _):
        def zj(j, _):
            ref[i, pl.ds(j * 16, 16)] = zv
            return 0
        return lax.fori_loop(0, width // 16, zj, 0)

    lax.fori_loop(0, rows, zi, 0)


def _sc_edge_sum(N, E, DE):
    EW = E // NW
    NCH = EW // CH
    RPT = N // NS

    @functools.partial(
        pl.kernel,
        out_type=jax.ShapeDtypeStruct((NC, N, DE), jnp.float32),
        mesh=_mesh(),
        compiler_params=pltpu.CompilerParams(use_tc_tiling_on_sc=False),
        scratch_types=[
            pltpu.VMEM((CH, DE), jnp.float32),
            pltpu.VMEM((CH,), jnp.int32),
            pltpu.VMEM((RPT, DE), jnp.float32),
            pltpu.VMEM_SHARED((N, DE), jnp.float32),
        ],
    )
    def k(ea_hbm, dst_hbm, out, rows, dstb, zb, acc):
        c = lax.axis_index("c")
        s = lax.axis_index("s")
        wid = c * NS + s
        _zero_vmem(zb, RPT, DE)
        pltpu.sync_copy(zb, acc.at[pl.ds(s * RPT, RPT)])
        plsc.subcore_barrier()

        def body(i, _):
            base = wid * EW + i * CH
            pltpu.sync_copy(dst_hbm.at[pl.ds(base, CH)], dstb)
            pltpu.sync_copy(ea_hbm.at[pl.ds(base, CH)], rows)
            pltpu.sync_copy(rows, acc.at[dstb], add=True)
            return 0

        lax.fori_loop(0, NCH, body, 0)
        plsc.subcore_barrier()
        pltpu.sync_copy(acc.at[pl.ds(s * RPT, RPT)],
                        out.at[c, pl.ds(s * RPT, RPT)])

    return k


def _sc_layer(N, E):
    EW = E // NW
    NCH = EW // CH
    GB = 25
    NG = NCH // GB
    RPT = N // NS
    RO = 125

    @functools.partial(
        pl.kernel,
        out_type=jax.ShapeDtypeStruct((2, NC, N, WH), jnp.float32),
        mesh=_mesh(),
        compiler_params=pltpu.CompilerParams(use_tc_tiling_on_sc=False),
        scratch_types=[
            pltpu.VMEM((CH, WH), jnp.float32),
            pltpu.VMEM((CH, WH), jnp.float32),
            pltpu.VMEM((GB, CH), jnp.int32),
            pltpu.VMEM((GB, CH), jnp.int32),
            pltpu.VMEM_SHARED((N, WH), jnp.float32),
            pltpu.SemaphoreType.DMA,
            pltpu.SemaphoreType.DMA,
        ],
    )
    def k(hA, hB, src_hbm, dst_hbm, out, r0, r1, src2d, dst2d, acc,
          g0, g1):
        c = lax.axis_index("c")
        s = lax.axis_index("s")
        wid = c * NS + s
        for p in range(2):
            h = hA if p == 0 else hB
            _zero_vmem(r0, CH, WH)
            for j in range(RPT // CH):
                pltpu.sync_copy(r0, acc.at[pl.ds(s * RPT + j * CH, CH)])
            rem = RPT - (RPT // CH) * CH
            if rem:
                pltpu.sync_copy(r0.at[pl.ds(0, rem)],
                                acc.at[pl.ds(s * RPT + RPT - rem, rem)])
            plsc.subcore_barrier()

            for g in range(NG):
                grow = wid * NCH + g * GB
                pltpu.sync_copy(src_hbm.at[pl.ds(grow, GB)], src2d)
                pltpu.sync_copy(dst_hbm.at[pl.ds(grow, GB)], dst2d)
                pltpu.async_copy(h.at[src2d.at[0]], r0, g0)

                def body(t, _):
                    j = 2 * t
                    pltpu.async_copy(h.at[src2d.at[j + 1]], r1, g1)
                    pltpu.make_async_copy(h.at[src2d.at[j]], r0, g0).wait()
                    pltpu.sync_copy(r0, acc.at[dst2d.at[j]], add=True)
                    pltpu.async_copy(h.at[src2d.at[j + 2]], r0, g0)
                    pltpu.make_async_copy(
                        h.at[src2d.at[j + 1]], r1, g1).wait()
                    pltpu.sync_copy(r1, acc.at[dst2d.at[j + 1]], add=True)
                    return 0

                lax.fori_loop(0, (GB - 1) // 2, body, 0)
                pltpu.make_async_copy(h.at[src2d.at[GB - 1]], r0, g0).wait()
                pltpu.sync_copy(r0, acc.at[dst2d.at[GB - 1]], add=True)

            plsc.subcore_barrier()
            for j in range(RPT // RO):
                r = s * RPT + j * RO
                pltpu.sync_copy(acc.at[pl.ds(r, RO)],
                                out.at[p, c, pl.ds(r, RO)])
            if p == 0:
                plsc.subcore_barrier()

    return k


def _tc_pre(N, R, DF, EMB, DE):
    def body(x_ref, win_ref, bin_ref, ap_ref, hA_ref, hB_ref, asum_ref):
        h = jnp.dot(x_ref[...], win_ref[...],
                    preferred_element_type=jnp.float32) + bin_ref[...]
        hA_ref[...] = h[:, :WH]
        hB_ref[...] = jnp.concatenate(
            [h[:, WH:], jnp.zeros((R, 2 * WH - EMB), jnp.float32)], axis=1)
        ap = ap_ref[...]
        asum_ref[...] = ap[0] + ap[1]

    return pl.pallas_call(
        body,
        grid=(N // R,),
        in_specs=[
            pl.BlockSpec((R, DF), lambda i: (i, 0)),
            pl.BlockSpec((DF, EMB), lambda i: (0, 0)),
            pl.BlockSpec((1, EMB), lambda i: (0, 0)),
            pl.BlockSpec((NC, R, DE), lambda i: (0, i, 0)),
        ],
        out_specs=[
            pl.BlockSpec((R, WH), lambda i: (i, 0)),
            pl.BlockSpec((R, WH), lambda i: (i, 0)),
            pl.BlockSpec((R, DE), lambda i: (i, 0)),
        ],
        out_shape=[
            jax.ShapeDtypeStruct((N, WH), jnp.float32),
            jax.ShapeDtypeStruct((N, WH), jnp.float32),
            jax.ShapeDtypeStruct((N, DE), jnp.float32),
        ],
    )


def _tc_mlp(N, R, EMB, DE, last):
    def body(aggp_ref, asum_ref, we_ref, w1_ref, b1_ref, w2_ref, b2_ref,
             hA_ref, hB_ref):
        ap = aggp_ref[...]
        agg_a = ap[0, 0] + ap[0, 1]
        agg_b = ap[1, 0] + ap[1, 1]
        agg = jnp.concatenate([agg_a, agg_b[:, :EMB - WH]], axis=1)
        agg = agg + jnp.dot(asum_ref[...], we_ref[...],
                            preferred_element_type=jnp.float32)
        y = jnp.maximum(jnp.dot(agg, w1_ref[...],
                                preferred_element_type=jnp.float32)
                        + b1_ref[...], 0.0)
        z = jnp.dot(y, w2_ref[...],
                    preferred_element_type=jnp.float32) + b2_ref[...]
        h = z if last else jnp.maximum(z, 0.0)
        hA_ref[...] = h[:, :WH]
        hB_ref[...] = jnp.concatenate(
            [h[:, WH:], jnp.zeros((R, 2 * WH - EMB), jnp.float32)], axis=1)

    return pl.pallas_call(
        body,
        grid=(N // R,),
        in_specs=[
            pl.BlockSpec((2, NC, R, WH), lambda i: (0, 0, i, 0)),
            pl.BlockSpec((R, DE), lambda i: (i, 0)),
            pl.BlockSpec((DE, EMB), lambda i: (0, 0)),
            pl.BlockSpec((EMB, 2 * EMB), lambda i: (0, 0)),
            pl.BlockSpec((1, 2 * EMB), lambda i: (0, 0)),
            pl.BlockSpec((2 * EMB, EMB), lambda i: (0, 0)),
            pl.BlockSpec((1, EMB), lambda i: (0, 0)),
        ],
        out_specs=[
            pl.BlockSpec((R, WH), lambda i: (i, 0)),
            pl.BlockSpec((R, WH), lambda i: (i, 0)),
        ],
        out_shape=[
            jax.ShapeDtypeStruct((N, WH), jnp.float32),
            jax.ShapeDtypeStruct((N, WH), jnp.float32),
        ],
    )


def _tc_final(N, R, EMB, PROJ):
    def body(hA_ref, hB_ref, wp_ref, bp_ref, out_ref):
        h = jnp.concatenate([hA_ref[...], hB_ref[...][:, :EMB - WH]], axis=1)
        o = jnp.dot(h, wp_ref[...],
                    preferred_element_type=jnp.float32) + bp_ref[...]
        n = jnp.maximum(
            jnp.sqrt(jnp.sum(o * o, axis=1, keepdims=True)), 1e-12)
        out_ref[...] = o / n

    return pl.pallas_call(
        body,
        grid=(N // R,),
        in_specs=[
            pl.BlockSpec((R, WH), lambda i: (i, 0)),
            pl.BlockSpec((R, WH), lambda i: (i, 0)),
            pl.BlockSpec((EMB, PROJ), lambda i: (0, 0)),
            pl.BlockSpec((1, PROJ), lambda i: (0, 0)),
        ],
        out_specs=pl.BlockSpec((R, PROJ), lambda i: (i, 0)),
        out_shape=jax.ShapeDtypeStruct((N, PROJ), jnp.float32),
    )


def kernel(x, edge_index, edge_attr, batch, W_in, b_in, W_edge, W1, b1,
           W2, b2, Wp, bp):
    N, DF = x.shape
    E = edge_index.shape[1]
    DE = edge_attr.shape[1]
    L, _, EMB = W_edge.shape
    PROJ = Wp.shape[1]
    R = 1000
    assert E % (NW * CH) == 0 and N % (NS * 125) == 0 and N % R == 0

    src = edge_index[0]
    dst = edge_index[1]
    src2 = src.reshape(E // CH, CH)
    dst2 = dst.reshape(E // CH, CH)

    ap = _sc_edge_sum(N, E, DE)(edge_attr, dst)
    hA, hB, asum = _tc_pre(N, R, DF, EMB, DE)(
        x, W_in, b_in.reshape(1, EMB), ap)

    layer_sc = _sc_layer(N, E)
    for l in range(L):
        parts = layer_sc(hA, hB, src2, dst2)
        hA, hB = _tc_mlp(N, R, EMB, DE, last=(l == L - 1))(
            parts, asum, W_edge[l], W1[l], b1[l].reshape(1, 2 * EMB),
            W2[l], b2[l].reshape(1, EMB))

    return _tc_final(N, R, EMB, PROJ)(hA, hB, Wp, bp.reshape(1, PROJ))


# speedup vs baseline: 7.5875x; 1.0845x over previous
import functools

import jax
import jax.numpy as jnp
from jax import lax
from jax.experimental import pallas as pl
from jax.experimental.pallas import tpu as pltpu
from jax.experimental.pallas import tpu_sc as plsc

NC = 2
NS = 16
NW = NC * NS
CH = 80
WH = 160


def _mesh():
    return plsc.VectorSubcoreMesh(
        core_axis_name="c", subcore_axis_name="s",
        num_cores=NC, num_subcores=NS)


def _zero_vmem(ref, rows, width):
    zv = jnp.zeros((16,), jnp.float32)

    def zi(i, _):
        def zj(j, _):
            ref[i, pl.ds(j * 16, 16)] = zv
            return 0
        return lax.fori_loop(0, width // 16, zj, 0)

    lax.fori_loop(0, rows, zi, 0)


def _sc_edge_sum(N, E, DE):
    EW = E // NW
    NCH = EW // CH
    RPT = N // NS

    @functools.partial(
        pl.kernel,
        out_type=jax.ShapeDtypeStruct((NC, N, DE), jnp.float32),
        mesh=_mesh(),
        compiler_params=pltpu.CompilerParams(use_tc_tiling_on_sc=False),
        scratch_types=[
            pltpu.VMEM((CH, DE), jnp.float32),
            pltpu.VMEM((CH,), jnp.int32),
            pltpu.VMEM((RPT, DE), jnp.float32),
            pltpu.VMEM_SHARED((N, DE), jnp.float32),
        ],
    )
    def k(ea_hbm, dst_hbm, out, rows, dstb, zb, acc):
        c = lax.axis_index("c")
        s = lax.axis_index("s")
        wid = c * NS + s
        _zero_vmem(zb, RPT, DE)
        pltpu.sync_copy(zb, acc.at[pl.ds(s * RPT, RPT)])
        plsc.subcore_barrier()

        def body(i, _):
            base = wid * EW + i * CH
            pltpu.sync_copy(dst_hbm.at[pl.ds(base, CH)], dstb)
            pltpu.sync_copy(ea_hbm.at[pl.ds(base, CH)], rows)
            pltpu.sync_copy(rows, acc.at[dstb], add=True)
            return 0

        lax.fori_loop(0, NCH, body, 0)
        plsc.subcore_barrier()
        pltpu.sync_copy(acc.at[pl.ds(s * RPT, RPT)],
                        out.at[c, pl.ds(s * RPT, RPT)])

    return k


def _sc_layer(N, E):
    EW = E // NW
    NCH = EW // CH
    GB = 25
    NG = NCH // GB
    RPT = N // NS
    RO = 125

    @functools.partial(
        pl.kernel,
        out_type=jax.ShapeDtypeStruct((2, NC, N, WH), jnp.float32),
        mesh=_mesh(),
        compiler_params=pltpu.CompilerParams(use_tc_tiling_on_sc=False),
        scratch_types=[
            pltpu.VMEM((CH, WH), jnp.float32),
            pltpu.VMEM((CH, WH), jnp.float32),
            pltpu.VMEM((GB, CH), jnp.int32),
            pltpu.VMEM((GB, CH), jnp.int32),
            pltpu.VMEM_SHARED((N, WH), jnp.float32),
            pltpu.SemaphoreType.DMA,
            pltpu.SemaphoreType.DMA,
        ],
    )
    def k(hA, hB, src_hbm, dst_hbm, out, r0, r1, src2d, dst2d, acc,
          g0, g1):
        c = lax.axis_index("c")
        s = lax.axis_index("s")
        wid = c * NS + s
        _zero_vmem(r0, CH, WH)
        for j in range(RPT // CH):
            pltpu.sync_copy(r0, acc.at[pl.ds(s * RPT + j * CH, CH)])
        rem = RPT - (RPT // CH) * CH
        if rem:
            pltpu.sync_copy(r0.at[pl.ds(0, rem)],
                            acc.at[pl.ds(s * RPT + RPT - rem, rem)])
        plsc.subcore_barrier()
        for p in range(2):
            h = hA if p == 0 else hB
            for g in range(NG):
                grow = wid * NCH + g * GB
                pltpu.async_copy(src_hbm.at[pl.ds(grow, GB)], src2d, g0)
                pltpu.sync_copy(dst_hbm.at[pl.ds(grow, GB)], dst2d)
                pltpu.make_async_copy(
                    src_hbm.at[pl.ds(grow, GB)], src2d, g0).wait()
                pltpu.async_copy(h.at[src2d.at[0]], r0, g0)

                def body(t, _):
                    j = 2 * t
                    pltpu.async_copy(h.at[src2d.at[j + 1]], r1, g1)
                    pltpu.make_async_copy(h.at[src2d.at[j]], r0, g0).wait()
                    pltpu.sync_copy(r0, acc.at[dst2d.at[j]], add=True)
                    pltpu.async_copy(h.at[src2d.at[j + 2]], r0, g0)
                    pltpu.make_async_copy(
                        h.at[src2d.at[j + 1]], r1, g1).wait()
                    pltpu.sync_copy(r1, acc.at[dst2d.at[j + 1]], add=True)
                    return 0

                lax.fori_loop(0, (GB - 1) // 2, body, 0)
                pltpu.make_async_copy(h.at[src2d.at[GB - 1]], r0, g0).wait()
                pltpu.sync_copy(r0, acc.at[dst2d.at[GB - 1]], add=True)

            plsc.subcore_barrier()
            for j in range(RPT // RO):
                r = s * RPT + j * RO
                pltpu.sync_copy(acc.at[pl.ds(r, RO)],
                                out.at[p, c, pl.ds(r, RO)])
            if p == 0:
                plsc.subcore_barrier()

    return k


def _tc_pre(N, R, DF, EMB):
    def body(x_ref, win_ref, bin_ref, hA_ref, hB_ref):
        h = jnp.dot(x_ref[...], win_ref[...],
                    preferred_element_type=jnp.float32) + bin_ref[...]
        hA_ref[...] = h[:, :WH]
        hB_ref[...] = jnp.concatenate(
            [h[:, WH:], jnp.zeros((R, 2 * WH - EMB), jnp.float32)], axis=1)

    return pl.pallas_call(
        body,
        grid=(N // R,),
        in_specs=[
            pl.BlockSpec((R, DF), lambda i: (i, 0)),
            pl.BlockSpec((DF, EMB), lambda i: (0, 0)),
            pl.BlockSpec((1, EMB), lambda i: (0, 0)),
        ],
        out_specs=[
            pl.BlockSpec((R, WH), lambda i: (i, 0)),
            pl.BlockSpec((R, WH), lambda i: (i, 0)),
        ],
        out_shape=[
            jax.ShapeDtypeStruct((N, WH), jnp.float32),
            jax.ShapeDtypeStruct((N, WH), jnp.float32),
        ],
    )


def _tc_mlp(N, R, EMB, DE, PROJ, last):
    def body(aggp_ref, ap_ref, we_ref, w1_ref, b1_ref, w2_ref, b2_ref,
             *rest):
        ap = aggp_ref[...]
        agg_a = ap[0, 0] + ap[0, 1]
        agg_b = (ap[1, 0] - ap[0, 0]) + (ap[1, 1] - ap[0, 1])
        agg = jnp.concatenate([agg_a, agg_b[:, :EMB - WH]], axis=1)
        ep = ap_ref[...]
        agg = agg + jnp.dot(ep[0] + ep[1], we_ref[...],
                            preferred_element_type=jnp.float32)
        y = jnp.maximum(jnp.dot(agg, w1_ref[...],
                                preferred_element_type=jnp.float32)
                        + b1_ref[...], 0.0)
        z = jnp.dot(y, w2_ref[...],
                    preferred_element_type=jnp.float32) + b2_ref[...]
        if last:
            wp_ref, bp_ref, out_ref = rest
            o = jnp.dot(z, wp_ref[...],
                        preferred_element_type=jnp.float32) + bp_ref[...]
            n = jnp.maximum(
                jnp.sqrt(jnp.sum(o * o, axis=1, keepdims=True)), 1e-12)
            out_ref[...] = o / n
        else:
            hA_ref, hB_ref = rest
            h = jnp.maximum(z, 0.0)
            hA_ref[...] = h[:, :WH]
            hB_ref[...] = jnp.concatenate(
                [h[:, WH:], jnp.zeros((R, 2 * WH - EMB), jnp.float32)],
                axis=1)

    in_specs = [
        pl.BlockSpec((2, NC, R, WH), lambda i: (0, 0, i, 0)),
        pl.BlockSpec((NC, R, DE), lambda i: (0, i, 0)),
        pl.BlockSpec((DE, EMB), lambda i: (0, 0)),
        pl.BlockSpec((EMB, 2 * EMB), lambda i: (0, 0)),
        pl.BlockSpec((1, 2 * EMB), lambda i: (0, 0)),
        pl.BlockSpec((2 * EMB, EMB), lambda i: (0, 0)),
        pl.BlockSpec((1, EMB), lambda i: (0, 0)),
    ]
    if last:
        in_specs += [
            pl.BlockSpec((EMB, PROJ), lambda i: (0, 0)),
            pl.BlockSpec((1, PROJ), lambda i: (0, 0)),
        ]
        out_specs = pl.BlockSpec((R, PROJ), lambda i: (i, 0))
        out_shape = jax.ShapeDtypeStruct((N, PROJ), jnp.float32)
    else:
        out_specs = [
            pl.BlockSpec((R, WH), lambda i: (i, 0)),
            pl.BlockSpec((R, WH), lambda i: (i, 0)),
        ]
        out_shape = [
            jax.ShapeDtypeStruct((N, WH), jnp.float32),
            jax.ShapeDtypeStruct((N, WH), jnp.float32),
        ]
    return pl.pallas_call(
        body,
        grid=(N // R,),
        in_specs=in_specs,
        out_specs=out_specs,
        out_shape=out_shape,
    )


def kernel(x, edge_index, edge_attr, batch, W_in, b_in, W_edge, W1, b1,
           W2, b2, Wp, bp):
    N, DF = x.shape
    E = edge_index.shape[1]
    DE = edge_attr.shape[1]
    L, _, EMB = W_edge.shape
    PROJ = Wp.shape[1]
    R = 1000
    assert E % (NW * CH) == 0 and N % (NS * 125) == 0 and N % R == 0

    src = edge_index[0]
    dst = edge_index[1]
    src2 = src.reshape(E // CH, CH)
    dst2 = dst.reshape(E // CH, CH)

    ap = _sc_edge_sum(N, E, DE)(edge_attr, dst)
    hA, hB = _tc_pre(N, R, DF, EMB)(x, W_in, b_in.reshape(1, EMB))

    layer_sc = _sc_layer(N, E)
    for l in range(L):
        parts = layer_sc(hA, hB, src2, dst2)
        args = (parts, ap, W_edge[l], W1[l], b1[l].reshape(1, 2 * EMB),
                W2[l], b2[l].reshape(1, EMB))
        if l < L - 1:
            hA, hB = _tc_mlp(N, R, EMB, DE, PROJ, last=False)(*args)
        else:
            return _tc_mlp(N, R, EMB, DE, PROJ, last=True)(
                *args, Wp, bp.reshape(1, PROJ))


# speedup vs baseline: 8.0497x; 1.0609x over previous
import functools

import jax
import jax.numpy as jnp
from jax import lax
from jax.experimental import pallas as pl
from jax.experimental.pallas import tpu as pltpu
from jax.experimental.pallas import tpu_sc as plsc

NC = 2
NS = 16
NW = NC * NS
CH = 80
WH = 160


def _mesh():
    return plsc.VectorSubcoreMesh(
        core_axis_name="c", subcore_axis_name="s",
        num_cores=NC, num_subcores=NS)


def _zero_vmem(ref, rows, width):
    zv = jnp.zeros((16,), jnp.float32)

    def zi(i, _):
        def zj(j, _):
            ref[i, pl.ds(j * 16, 16)] = zv
            return 0
        return lax.fori_loop(0, width // 16, zj, 0)

    lax.fori_loop(0, rows, zi, 0)


def _sc_edge_sum(N, E, DE):
    EW = E // NW
    NCH = EW // CH
    GB = 25
    NG = NCH // GB
    GE = GB * CH
    RPT = N // NS

    @functools.partial(
        pl.kernel,
        out_type=jax.ShapeDtypeStruct((NC, N, DE), jnp.float32),
        mesh=_mesh(),
        compiler_params=pltpu.CompilerParams(use_tc_tiling_on_sc=False),
        scratch_types=[
            pltpu.VMEM((GE, DE), jnp.float32),
            pltpu.VMEM((GE, DE), jnp.float32),
            pltpu.VMEM((GB, CH), jnp.int32),
            pltpu.VMEM((GB, CH), jnp.int32),
            pltpu.VMEM((RPT, DE), jnp.float32),
            pltpu.VMEM_SHARED((N, DE), jnp.float32),
            pltpu.SemaphoreType.DMA,
            pltpu.SemaphoreType.DMA,
        ],
    )
    def k(ea_hbm, dst_hbm, out, ea0, ea1, d0, d1, zb, acc, isem, ssem):
        c = lax.axis_index("c")
        s = lax.axis_index("s")
        wid = c * NS + s
        _zero_vmem(zb, RPT, DE)
        pltpu.sync_copy(zb, acc.at[pl.ds(s * RPT, RPT)])
        plsc.subcore_barrier()

        eab = (ea0, ea1)
        db = (d0, d1)
        pltpu.sync_copy(ea_hbm.at[pl.ds(wid * EW, GE)], ea0)
        pltpu.sync_copy(dst_hbm.at[pl.ds(wid * NCH, GB)], d0)
        for g in range(NG):
            ea, dst2d = eab[g % 2], db[g % 2]
            if g < NG - 1:
                base = wid * EW + (g + 1) * GE
                pltpu.async_copy(ea_hbm.at[pl.ds(base, GE)],
                                 eab[(g + 1) % 2], isem)
                pltpu.async_copy(dst_hbm.at[pl.ds(wid * NCH + (g + 1) * GB,
                                                  GB)],
                                 db[(g + 1) % 2], isem)

            def fire(j, _):
                pltpu.async_copy(ea.at[pl.ds(j * CH, CH)],
                                 acc.at[dst2d.at[j]], ssem, add=True)
                return 0

            lax.fori_loop(0, GB, fire, 0)

            def drain(j, _):
                pltpu.make_async_copy(ea.at[pl.ds(0, CH)],
                                      acc.at[dst2d.at[0]], ssem).wait()
                return 0

            lax.fori_loop(0, GB, drain, 0)
            if g < NG - 1:
                pltpu.make_async_copy(
                    ea_hbm.at[pl.ds(wid * EW, GE)],
                    eab[(g + 1) % 2], isem).wait()
                pltpu.make_async_copy(
                    dst_hbm.at[pl.ds(wid * NCH, GB)],
                    db[(g + 1) % 2], isem).wait()

        plsc.subcore_barrier()
        pltpu.sync_copy(acc.at[pl.ds(s * RPT, RPT)],
                        out.at[c, pl.ds(s * RPT, RPT)])

    return k


def _sc_layer(N, E):
    EW = E // NW
    NCH = EW // CH
    GB = 25
    NG = NCH // GB
    RPT = N // NS
    RO = 125

    @functools.partial(
        pl.kernel,
        out_type=jax.ShapeDtypeStruct((2, NC, N, WH), jnp.float32),
        mesh=_mesh(),
        compiler_params=pltpu.CompilerParams(use_tc_tiling_on_sc=False),
        scratch_types=[
            pltpu.VMEM((CH, WH), jnp.float32),
            pltpu.VMEM((CH, WH), jnp.float32),
            pltpu.VMEM((GB, CH), jnp.int32),
            pltpu.VMEM((GB, CH), jnp.int32),
            pltpu.VMEM_SHARED((N, WH), jnp.float32),
            pltpu.SemaphoreType.DMA,
            pltpu.SemaphoreType.DMA,
        ],
    )
    def k(hA, hB, src_hbm, dst_hbm, out, r0, r1, src2d, dst2d, acc,
          g0, g1):
        c = lax.axis_index("c")
        s = lax.axis_index("s")
        wid = c * NS + s
        _zero_vmem(r0, CH, WH)
        for j in range(RPT // CH):
            pltpu.sync_copy(r0, acc.at[pl.ds(s * RPT + j * CH, CH)])
        rem = RPT - (RPT // CH) * CH
        if rem:
            pltpu.sync_copy(r0.at[pl.ds(0, rem)],
                            acc.at[pl.ds(s * RPT + RPT - rem, rem)])
        plsc.subcore_barrier()
        for p in range(2):
            h = hA if p == 0 else hB
            for g in range(NG):
                grow = wid * NCH + g * GB
                pltpu.async_copy(src_hbm.at[pl.ds(grow, GB)], src2d, g0)
                pltpu.sync_copy(dst_hbm.at[pl.ds(grow, GB)], dst2d)
                pltpu.make_async_copy(
                    src_hbm.at[pl.ds(grow, GB)], src2d, g0).wait()
                pltpu.async_copy(h.at[src2d.at[0]], r0, g0)

                def body(t, _):
                    j = 2 * t
                    pltpu.async_copy(h.at[src2d.at[j + 1]], r1, g1)
                    pltpu.make_async_copy(h.at[src2d.at[j]], r0, g0).wait()
                    pltpu.sync_copy(r0, acc.at[dst2d.at[j]], add=True)
                    pltpu.async_copy(h.at[src2d.at[j + 2]], r0, g0)
                    pltpu.make_async_copy(
                        h.at[src2d.at[j + 1]], r1, g1).wait()
                    pltpu.sync_copy(r1, acc.at[dst2d.at[j + 1]], add=True)
                    return 0

                lax.fori_loop(0, (GB - 1) // 2, body, 0)
                pltpu.make_async_copy(h.at[src2d.at[GB - 1]], r0, g0).wait()
                pltpu.sync_copy(r0, acc.at[dst2d.at[GB - 1]], add=True)

            plsc.subcore_barrier()
            for j in range(RPT // RO):
                r = s * RPT + j * RO
                pltpu.sync_copy(acc.at[pl.ds(r, RO)],
                                out.at[p, c, pl.ds(r, RO)])
            if p == 0:
                plsc.subcore_barrier()

    return k


def _tc_pre(N, R, DF, EMB):
    def body(x_ref, win_ref, bin_ref, hA_ref, hB_ref):
        h = jnp.dot(x_ref[...], win_ref[...],
                    preferred_element_type=jnp.float32) + bin_ref[...]
        hA_ref[...] = h[:, :WH]
        hB_ref[...] = jnp.concatenate(
            [h[:, WH:], jnp.zeros((R, 2 * WH - EMB), jnp.float32)], axis=1)

    return pl.pallas_call(
        body,
        grid=(N // R,),
        in_specs=[
            pl.BlockSpec((R, DF), lambda i: (i, 0)),
            pl.BlockSpec((DF, EMB), lambda i: (0, 0)),
            pl.BlockSpec((1, EMB), lambda i: (0, 0)),
        ],
        out_specs=[
            pl.BlockSpec((R, WH), lambda i: (i, 0)),
            pl.BlockSpec((R, WH), lambda i: (i, 0)),
        ],
        out_shape=[
            jax.ShapeDtypeStruct((N, WH), jnp.float32),
            jax.ShapeDtypeStruct((N, WH), jnp.float32),
        ],
    )


def _tc_mlp(N, R, EMB, DE, PROJ, last):
    def body(aggp_ref, ap_ref, we_ref, w1_ref, b1_ref, w2_ref, b2_ref,
             *rest):
        ap = aggp_ref[...]
        agg_a = ap[0, 0] + ap[0, 1]
        agg_b = (ap[1, 0] - ap[0, 0]) + (ap[1, 1] - ap[0, 1])
        agg = jnp.concatenate([agg_a, agg_b[:, :EMB - WH]], axis=1)
        ep = ap_ref[...]
        agg = agg + jnp.dot(ep[0] + ep[1], we_ref[...],
                            preferred_element_type=jnp.float32)
        y = jnp.maximum(jnp.dot(agg, w1_ref[...],
                                preferred_element_type=jnp.float32)
                        + b1_ref[...], 0.0)
        z = jnp.dot(y, w2_ref[...],
                    preferred_element_type=jnp.float32) + b2_ref[...]
        if last:
            wp_ref, bp_ref, out_ref = rest
            o = jnp.dot(z, wp_ref[...],
                        preferred_element_type=jnp.float32) + bp_ref[...]
            n = jnp.maximum(
                jnp.sqrt(jnp.sum(o * o, axis=1, keepdims=True)), 1e-12)
            out_ref[...] = o / n
        else:
            hA_ref, hB_ref = rest
            h = jnp.maximum(z, 0.0)
            hA_ref[...] = h[:, :WH]
            hB_ref[...] = jnp.concatenate(
                [h[:, WH:], jnp.zeros((R, 2 * WH - EMB), jnp.float32)],
                axis=1)

    in_specs = [
        pl.BlockSpec((2, NC, R, WH), lambda i: (0, 0, i, 0)),
        pl.BlockSpec((NC, R, DE), lambda i: (0, i, 0)),
        pl.BlockSpec((DE, EMB), lambda i: (0, 0)),
        pl.BlockSpec((EMB, 2 * EMB), lambda i: (0, 0)),
        pl.BlockSpec((1, 2 * EMB), lambda i: (0, 0)),
        pl.BlockSpec((2 * EMB, EMB), lambda i: (0, 0)),
        pl.BlockSpec((1, EMB), lambda i: (0, 0)),
    ]
    if last:
        in_specs += [
            pl.BlockSpec((EMB, PROJ), lambda i: (0, 0)),
            pl.BlockSpec((1, PROJ), lambda i: (0, 0)),
        ]
        out_specs = pl.BlockSpec((R, PROJ), lambda i: (i, 0))
        out_shape = jax.ShapeDtypeStruct((N, PROJ), jnp.float32)
    else:
        out_specs = [
            pl.BlockSpec((R, WH), lambda i: (i, 0)),
            pl.BlockSpec((R, WH), lambda i: (i, 0)),
        ]
        out_shape = [
            jax.ShapeDtypeStruct((N, WH), jnp.float32),
            jax.ShapeDtypeStruct((N, WH), jnp.float32),
        ]
    return pl.pallas_call(
        body,
        grid=(N // R,),
        in_specs=in_specs,
        out_specs=out_specs,
        out_shape=out_shape,
    )


def kernel(x, edge_index, edge_attr, batch, W_in, b_in, W_edge, W1, b1,
           W2, b2, Wp, bp):
    N, DF = x.shape
    E = edge_index.shape[1]
    DE = edge_attr.shape[1]
    L, _, EMB = W_edge.shape
    PROJ = Wp.shape[1]
    R = 1000
    assert E % (NW * CH) == 0 and N % (NS * 125) == 0 and N % R == 0

    src = edge_index[0]
    dst = edge_index[1]
    src2 = src.reshape(E // CH, CH)
    dst2 = dst.reshape(E // CH, CH)

    ap = _sc_edge_sum(N, E, DE)(edge_attr, dst2)
    hA, hB = _tc_pre(N, R, DF, EMB)(x, W_in, b_in.reshape(1, EMB))

    layer_sc = _sc_layer(N, E)
    for l in range(L):
        parts = layer_sc(hA, hB, src2, dst2)
        args = (parts, ap, W_edge[l], W1[l], b1[l].reshape(1, 2 * EMB),
                W2[l], b2[l].reshape(1, EMB))
        if l < L - 1:
            hA, hB = _tc_mlp(N, R, EMB, DE, PROJ, last=False)(*args)
        else:
            return _tc_mlp(N, R, EMB, DE, PROJ, last=True)(
                *args, Wp, bp.reshape(1, PROJ))


# speedup vs baseline: 8.1643x; 1.0142x over previous
import functools

import jax
import jax.numpy as jnp
from jax import lax
from jax.experimental import pallas as pl
from jax.experimental.pallas import tpu as pltpu
from jax.experimental.pallas import tpu_sc as plsc

NC = 2
NS = 16
NW = NC * NS
CH = 80
WH = 160


def _mesh():
    return plsc.VectorSubcoreMesh(
        core_axis_name="c", subcore_axis_name="s",
        num_cores=NC, num_subcores=NS)


def _zero_vmem(ref, rows, width):
    zv = jnp.zeros((16,), jnp.float32)

    def zi(i, _):
        def zj(j, _):
            ref[i, pl.ds(j * 16, 16)] = zv
            return 0
        return lax.fori_loop(0, width // 16, zj, 0)

    lax.fori_loop(0, rows, zi, 0)


def _sc_edge_sum(N, E, DE):
    EW = E // NW
    NCH = EW // CH
    GB = 25
    NG = NCH // GB
    GE = GB * CH
    RPT = N // NS

    @functools.partial(
        pl.kernel,
        out_type=jax.ShapeDtypeStruct((NC, N, DE), jnp.float32),
        mesh=_mesh(),
        compiler_params=pltpu.CompilerParams(use_tc_tiling_on_sc=False),
        scratch_types=[
            pltpu.VMEM((GE, DE), jnp.float32),
            pltpu.VMEM((GE, DE), jnp.float32),
            pltpu.VMEM((GB, CH), jnp.int32),
            pltpu.VMEM((GB, CH), jnp.int32),
            pltpu.VMEM((RPT, DE), jnp.float32),
            pltpu.VMEM_SHARED((N, DE), jnp.float32),
            pltpu.SemaphoreType.DMA,
            pltpu.SemaphoreType.DMA,
        ],
    )
    def k(ea_hbm, dst_hbm, out, ea0, ea1, d0, d1, zb, acc, isem, ssem):
        c = lax.axis_index("c")
        s = lax.axis_index("s")
        wid = c * NS + s
        _zero_vmem(zb, RPT, DE)
        pltpu.sync_copy(zb, acc.at[pl.ds(s * RPT, RPT)])
        plsc.subcore_barrier()

        eab = (ea0, ea1)
        db = (d0, d1)
        pltpu.sync_copy(ea_hbm.at[pl.ds(wid * EW, GE)], ea0)
        pltpu.sync_copy(dst_hbm.at[pl.ds(wid * NCH, GB)], d0)
        for g in range(NG):
            ea, dst2d = eab[g % 2], db[g % 2]
            if g < NG - 1:
                base = wid * EW + (g + 1) * GE
                pltpu.async_copy(ea_hbm.at[pl.ds(base, GE)],
                                 eab[(g + 1) % 2], isem)
                pltpu.async_copy(dst_hbm.at[pl.ds(wid * NCH + (g + 1) * GB,
                                                  GB)],
                                 db[(g + 1) % 2], isem)

            def fire(j, _):
                pltpu.async_copy(ea.at[pl.ds(j * CH, CH)],
                                 acc.at[dst2d.at[j]], ssem, add=True)
                return 0

            lax.fori_loop(0, GB, fire, 0)

            def drain(j, _):
                pltpu.make_async_copy(ea.at[pl.ds(0, CH)],
                                      acc.at[dst2d.at[0]], ssem).wait()
                return 0

            lax.fori_loop(0, GB, drain, 0)
            if g < NG - 1:
                pltpu.make_async_copy(
                    ea_hbm.at[pl.ds(wid * EW, GE)],
                    eab[(g + 1) % 2], isem).wait()
                pltpu.make_async_copy(
                    dst_hbm.at[pl.ds(wid * NCH, GB)],
                    db[(g + 1) % 2], isem).wait()

        plsc.subcore_barrier()
        pltpu.sync_copy(acc.at[pl.ds(s * RPT, RPT)],
                        out.at[c, pl.ds(s * RPT, RPT)])

    return k


def _sc_layer(N, E):
    EW = E // NW
    NCH = EW // CH
    GB = 25
    NG = NCH // GB
    RPT = N // NS
    RO = 125

    @functools.partial(
        pl.kernel,
        out_type=jax.ShapeDtypeStruct((2, NC, N, WH), jnp.float32),
        mesh=_mesh(),
        compiler_params=pltpu.CompilerParams(use_tc_tiling_on_sc=False),
        scratch_types=[
            pltpu.VMEM((CH, WH), jnp.float32),
            pltpu.VMEM((CH, WH), jnp.float32),
            pltpu.VMEM((GB, CH), jnp.int32),
            pltpu.VMEM((GB, CH), jnp.int32),
            pltpu.VMEM((CH,), jnp.int32),
            pltpu.VMEM((CH,), jnp.int32),
            pltpu.VMEM((CH,), jnp.int32),
            pltpu.VMEM((CH,), jnp.int32),
            pltpu.VMEM_SHARED((N, WH), jnp.float32),
            pltpu.SemaphoreType.DMA,
            pltpu.SemaphoreType.DMA,
            pltpu.SemaphoreType.DMA,
        ],
    )
    def k(hA, hB, src_hbm, dst_hbm, out, r0, r1, src2d, dst2d,
          s23, s24, d23, d24, acc, g0, g1, isem):
        c = lax.axis_index("c")
        s = lax.axis_index("s")
        wid = c * NS + s

        def fire_idx(grow):
            pltpu.async_copy(src_hbm.at[pl.ds(grow, GB)], src2d, isem)
            pltpu.async_copy(dst_hbm.at[pl.ds(grow, GB)], dst2d, isem)

        def wait_idx(grow):
            pltpu.make_async_copy(
                src_hbm.at[pl.ds(grow, GB)], src2d, isem).wait()
            pltpu.make_async_copy(
                dst_hbm.at[pl.ds(grow, GB)], dst2d, isem).wait()

        fire_idx(wid * NCH)
        _zero_vmem(r0, CH, WH)
        for j in range(RPT // CH):
            pltpu.sync_copy(r0, acc.at[pl.ds(s * RPT + j * CH, CH)])
        rem = RPT - (RPT // CH) * CH
        if rem:
            pltpu.sync_copy(r0.at[pl.ds(0, rem)],
                            acc.at[pl.ds(s * RPT + RPT - rem, rem)])
        plsc.subcore_barrier()
        for p in range(2):
            h = hA if p == 0 else hB
            for g in range(NG):
                wait_idx(wid * NCH + g * GB)
                for kk in range(CH // 16):
                    sl = pl.ds(kk * 16, 16)
                    s23[sl] = src2d[GB - 2, sl]
                    s24[sl] = src2d[GB - 1, sl]
                    d23[sl] = dst2d[GB - 2, sl]
                    d24[sl] = dst2d[GB - 1, sl]
                pltpu.async_copy(h.at[src2d.at[0]], r0, g0)

                def body(t, _):
                    j = 2 * t
                    pltpu.async_copy(h.at[src2d.at[j + 1]], r1, g1)
                    pltpu.make_async_copy(h.at[src2d.at[j]], r0, g0).wait()
                    pltpu.sync_copy(r0, acc.at[dst2d.at[j]], add=True)
                    pltpu.async_copy(h.at[src2d.at[j + 2]], r0, g0)
                    pltpu.make_async_copy(
                        h.at[src2d.at[j + 1]], r1, g1).wait()
                    pltpu.sync_copy(r1, acc.at[dst2d.at[j + 1]], add=True)
                    return 0

                lax.fori_loop(0, (GB - 3) // 2, body, 0)
                pltpu.async_copy(h.at[s23], r1, g1)
                pltpu.make_async_copy(
                    h.at[src2d.at[GB - 3]], r0, g0).wait()
                pltpu.sync_copy(r0, acc.at[dst2d.at[GB - 3]], add=True)
                nxt = p * NG + g + 1
                if nxt < 2 * NG:
                    fire_idx(wid * NCH + (nxt % NG) * GB)
                pltpu.async_copy(h.at[s24], r0, g0)
                pltpu.make_async_copy(h.at[s23], r1, g1).wait()
                pltpu.sync_copy(r1, acc.at[d23], add=True)
                pltpu.make_async_copy(h.at[s24], r0, g0).wait()
                pltpu.sync_copy(r0, acc.at[d24], add=True)

            plsc.subcore_barrier()
            for j in range(RPT // RO):
                r = s * RPT + j * RO
                pltpu.sync_copy(acc.at[pl.ds(r, RO)],
                                out.at[p, c, pl.ds(r, RO)])
            if p == 0:
                plsc.subcore_barrier()

    return k


def _tc_pre(N, R, DF, EMB):
    def body(x_ref, win_ref, bin_ref, hA_ref, hB_ref):
        h = jnp.dot(x_ref[...], win_ref[...],
                    preferred_element_type=jnp.float32) + bin_ref[...]
        hA_ref[...] = h[:, :WH]
        hB_ref[...] = jnp.concatenate(
            [h[:, WH:], jnp.zeros((R, 2 * WH - EMB), jnp.float32)], axis=1)

    return pl.pallas_call(
        body,
        grid=(N // R,),
        in_specs=[
            pl.BlockSpec((R, DF), lambda i: (i, 0)),
            pl.BlockSpec((DF, EMB), lambda i: (0, 0)),
            pl.BlockSpec((1, EMB), lambda i: (0, 0)),
        ],
        out_specs=[
            pl.BlockSpec((R, WH), lambda i: (i, 0)),
            pl.BlockSpec((R, WH), lambda i: (i, 0)),
        ],
        out_shape=[
            jax.ShapeDtypeStruct((N, WH), jnp.float32),
            jax.ShapeDtypeStruct((N, WH), jnp.float32),
        ],
    )


def _tc_mlp(N, R, EMB, DE, PROJ, last):
    def body(aggp_ref, ap_ref, we_ref, w1_ref, b1_ref, w2_ref, b2_ref,
             *rest):
        ap = aggp_ref[...]
        agg_a = ap[0, 0] + ap[0, 1]
        agg_b = (ap[1, 0] - ap[0, 0]) + (ap[1, 1] - ap[0, 1])
        agg = jnp.concatenate([agg_a, agg_b[:, :EMB - WH]], axis=1)
        ep = ap_ref[...]
        agg = agg + jnp.dot(ep[0] + ep[1], we_ref[...],
                            preferred_element_type=jnp.float32)
        y = jnp.maximum(jnp.dot(agg, w1_ref[...],
                                preferred_element_type=jnp.float32)
                        + b1_ref[...], 0.0)
        z = jnp.dot(y, w2_ref[...],
                    preferred_element_type=jnp.float32) + b2_ref[...]
        if last:
            wp_ref, bp_ref, out_ref = rest
            o = jnp.dot(z, wp_ref[...],
                        preferred_element_type=jnp.float32) + bp_ref[...]
            n = jnp.maximum(
                jnp.sqrt(jnp.sum(o * o, axis=1, keepdims=True)), 1e-12)
            out_ref[...] = o / n
        else:
            hA_ref, hB_ref = rest
            h = jnp.maximum(z, 0.0)
            hA_ref[...] = h[:, :WH]
            hB_ref[...] = jnp.concatenate(
                [h[:, WH:], jnp.zeros((R, 2 * WH - EMB), jnp.float32)],
                axis=1)

    in_specs = [
        pl.BlockSpec((2, NC, R, WH), lambda i: (0, 0, i, 0)),
        pl.BlockSpec((NC, R, DE), lambda i: (0, i, 0)),
        pl.BlockSpec((DE, EMB), lambda i: (0, 0)),
        pl.BlockSpec((EMB, 2 * EMB), lambda i: (0, 0)),
        pl.BlockSpec((1, 2 * EMB), lambda i: (0, 0)),
        pl.BlockSpec((2 * EMB, EMB), lambda i: (0, 0)),
        pl.BlockSpec((1, EMB), lambda i: (0, 0)),
    ]
    if last:
        in_specs += [
            pl.BlockSpec((EMB, PROJ), lambda i: (0, 0)),
            pl.BlockSpec((1, PROJ), lambda i: (0, 0)),
        ]
        out_specs = pl.BlockSpec((R, PROJ), lambda i: (i, 0))
        out_shape = jax.ShapeDtypeStruct((N, PROJ), jnp.float32)
    else:
        out_specs = [
            pl.BlockSpec((R, WH), lambda i: (i, 0)),
            pl.BlockSpec((R, WH), lambda i: (i, 0)),
        ]
        out_shape = [
            jax.ShapeDtypeStruct((N, WH), jnp.float32),
            jax.ShapeDtypeStruct((N, WH), jnp.float32),
        ]
    return pl.pallas_call(
        body,
        grid=(N // R,),
        in_specs=in_specs,
        out_specs=out_specs,
        out_shape=out_shape,
    )


def kernel(x, edge_index, edge_attr, batch, W_in, b_in, W_edge, W1, b1,
           W2, b2, Wp, bp):
    N, DF = x.shape
    E = edge_index.shape[1]
    DE = edge_attr.shape[1]
    L, _, EMB = W_edge.shape
    PROJ = Wp.shape[1]
    R = 1000
    assert E % (NW * CH) == 0 and N % (NS * 125) == 0 and N % R == 0

    src = edge_index[0]
    dst = edge_index[1]
    src2 = src.reshape(E // CH, CH)
    dst2 = dst.reshape(E // CH, CH)

    ap = _sc_edge_sum(N, E, DE)(edge_attr, dst2)
    hA, hB = _tc_pre(N, R, DF, EMB)(x, W_in, b_in.reshape(1, EMB))

    layer_sc = _sc_layer(N, E)
    for l in range(L):
        parts = layer_sc(hA, hB, src2, dst2)
        args = (parts, ap, W_edge[l], W1[l], b1[l].reshape(1, 2 * EMB),
                W2[l], b2[l].reshape(1, EMB))
        if l < L - 1:
            hA, hB = _tc_mlp(N, R, EMB, DE, PROJ, last=False)(*args)
        else:
            return _tc_mlp(N, R, EMB, DE, PROJ, last=True)(
                *args, Wp, bp.reshape(1, PROJ))


# speedup vs baseline: 8.1798x; 1.0019x over previous
import functools

import jax
import jax.numpy as jnp
from jax import lax
from jax.experimental import pallas as pl
from jax.experimental.pallas import tpu as pltpu
from jax.experimental.pallas import tpu_sc as plsc

NC = 2
NS = 16
NW = NC * NS
CH = 80
WH = 160


def _mesh():
    return plsc.VectorSubcoreMesh(
        core_axis_name="c", subcore_axis_name="s",
        num_cores=NC, num_subcores=NS)


def _zero_vmem(ref, rows, width):
    zv = jnp.zeros((16,), jnp.float32)

    def zi(i, _):
        def zj(j, _):
            ref[i, pl.ds(j * 16, 16)] = zv
            return 0
        return lax.fori_loop(0, width // 16, zj, 0)

    lax.fori_loop(0, rows, zi, 0)


def _sc_edge_sum(N, E, DE):
    EW = E // NW
    NCH = EW // CH
    GB = 25
    NG = NCH // GB
    GE = GB * CH
    RPT = N // NS

    @functools.partial(
        pl.kernel,
        out_type=jax.ShapeDtypeStruct((NC, N, DE), jnp.float32),
        mesh=_mesh(),
        compiler_params=pltpu.CompilerParams(use_tc_tiling_on_sc=False),
        scratch_types=[
            pltpu.VMEM((GE, DE), jnp.float32),
            pltpu.VMEM((GE, DE), jnp.float32),
            pltpu.VMEM((GB, CH), jnp.int32),
            pltpu.VMEM((GB, CH), jnp.int32),
            pltpu.VMEM((RPT, DE), jnp.float32),
            pltpu.VMEM_SHARED((N, DE), jnp.float32),
            pltpu.SemaphoreType.DMA,
            pltpu.SemaphoreType.DMA,
        ],
    )
    def k(ea_hbm, dst_hbm, out, ea0, ea1, d0, d1, zb, acc, isem, ssem):
        c = lax.axis_index("c")
        s = lax.axis_index("s")
        wid = c * NS + s
        _zero_vmem(zb, RPT, DE)
        pltpu.sync_copy(zb, acc.at[pl.ds(s * RPT, RPT)])
        plsc.subcore_barrier()

        eab = (ea0, ea1)
        db = (d0, d1)
        pltpu.sync_copy(ea_hbm.at[pl.ds(wid * EW, GE)], ea0)
        pltpu.sync_copy(dst_hbm.at[pl.ds(wid * NCH, GB)], d0)
        for g in range(NG):
            ea, dst2d = eab[g % 2], db[g % 2]
            if g < NG - 1:
                base = wid * EW + (g + 1) * GE
                pltpu.async_copy(ea_hbm.at[pl.ds(base, GE)],
                                 eab[(g + 1) % 2], isem)
                pltpu.async_copy(dst_hbm.at[pl.ds(wid * NCH + (g + 1) * GB,
                                                  GB)],
                                 db[(g + 1) % 2], isem)

            def fire(j, _):
                pltpu.async_copy(ea.at[pl.ds(j * CH, CH)],
                                 acc.at[dst2d.at[j]], ssem, add=True)
                return 0

            lax.fori_loop(0, GB, fire, 0)

            def drain(j, _):
                pltpu.make_async_copy(ea.at[pl.ds(0, CH)],
                                      acc.at[dst2d.at[0]], ssem).wait()
                return 0

            lax.fori_loop(0, GB, drain, 0)
            if g < NG - 1:
                pltpu.make_async_copy(
                    ea_hbm.at[pl.ds(wid * EW, GE)],
                    eab[(g + 1) % 2], isem).wait()
                pltpu.make_async_copy(
                    dst_hbm.at[pl.ds(wid * NCH, GB)],
                    db[(g + 1) % 2], isem).wait()

        plsc.subcore_barrier()
        pltpu.sync_copy(acc.at[pl.ds(s * RPT, RPT)],
                        out.at[c, pl.ds(s * RPT, RPT)])

    return k


def _sc_layer(N, E):
    EW = E // NW
    NCH = EW // CH
    GB = 25
    NG = NCH // GB
    RPT = N // NS
    RO = 125

    @functools.partial(
        pl.kernel,
        out_type=jax.ShapeDtypeStruct((2, NC, N, WH), jnp.float32),
        mesh=_mesh(),
        compiler_params=pltpu.CompilerParams(use_tc_tiling_on_sc=False),
        scratch_types=[
            pltpu.VMEM((CH, WH), jnp.float32),
            pltpu.VMEM((CH, WH), jnp.float32),
            pltpu.VMEM((GB, CH), jnp.int32),
            pltpu.VMEM((GB, CH), jnp.int32),
            pltpu.VMEM((CH,), jnp.int32),
            pltpu.VMEM((CH,), jnp.int32),
            pltpu.VMEM((CH,), jnp.int32),
            pltpu.VMEM((CH,), jnp.int32),
            pltpu.VMEM_SHARED((N, WH), jnp.float32),
            pltpu.SemaphoreType.DMA,
            pltpu.SemaphoreType.DMA,
            pltpu.SemaphoreType.DMA,
        ],
    )
    def k(hA, hB, src_hbm, dst_hbm, out, r0, r1, src2d, dst2d,
          s23, s24, d23, d24, acc, g0, g1, isem):
        c = lax.axis_index("c")
        s = lax.axis_index("s")
        wid = c * NS + s

        def fire_idx(grow):
            pltpu.async_copy(src_hbm.at[pl.ds(grow, GB)], src2d, isem)
            pltpu.async_copy(dst_hbm.at[pl.ds(grow, GB)], dst2d, isem)

        def wait_idx(grow):
            pltpu.make_async_copy(
                src_hbm.at[pl.ds(grow, GB)], src2d, isem).wait()
            pltpu.make_async_copy(
                dst_hbm.at[pl.ds(grow, GB)], dst2d, isem).wait()

        fire_idx(wid * NCH)
        _zero_vmem(r0, CH, WH)
        for j in range(RPT // CH):
            pltpu.sync_copy(r0, acc.at[pl.ds(s * RPT + j * CH, CH)])
        rem = RPT - (RPT // CH) * CH
        if rem:
            pltpu.sync_copy(r0.at[pl.ds(0, rem)],
                            acc.at[pl.ds(s * RPT + RPT - rem, rem)])
        plsc.subcore_barrier()
        for p in range(2):
            h = hA if p == 0 else hB
            for g in range(NG):
                wait_idx(wid * NCH + g * GB)
                for kk in range(CH // 16):
                    sl = pl.ds(kk * 16, 16)
                    s23[sl] = src2d[GB - 2, sl]
                    s24[sl] = src2d[GB - 1, sl]
                    d23[sl] = dst2d[GB - 2, sl]
                    d24[sl] = dst2d[GB - 1, sl]
                pltpu.async_copy(h.at[src2d.at[0]], r0, g0)

                def body(t, _):
                    j = 2 * t
                    pltpu.async_copy(h.at[src2d.at[j + 1]], r1, g1)
                    pltpu.make_async_copy(h.at[src2d.at[j]], r0, g0).wait()
                    pltpu.sync_copy(r0, acc.at[dst2d.at[j]], add=True)
                    pltpu.async_copy(h.at[src2d.at[j + 2]], r0, g0)
                    pltpu.make_async_copy(
                        h.at[src2d.at[j + 1]], r1, g1).wait()
                    pltpu.sync_copy(r1, acc.at[dst2d.at[j + 1]], add=True)
                    return 0

                lax.fori_loop(0, (GB - 3) // 2, body, 0)
                pltpu.async_copy(h.at[s23], r1, g1)
                pltpu.make_async_copy(
                    h.at[src2d.at[GB - 3]], r0, g0).wait()
                pltpu.sync_copy(r0, acc.at[dst2d.at[GB - 3]], add=True)
                nxt = p * NG + g + 1
                if nxt < 2 * NG:
                    fire_idx(wid * NCH + (nxt % NG) * GB)
                pltpu.async_copy(h.at[s24], r0, g0)
                pltpu.make_async_copy(h.at[s23], r1, g1).wait()
                pltpu.sync_copy(r1, acc.at[d23], add=True)
                pltpu.make_async_copy(h.at[s24], r0, g0).wait()
                pltpu.sync_copy(r0, acc.at[d24], add=True)

            plsc.subcore_barrier()
            for j in range(RPT // RO):
                r = s * RPT + j * RO
                pltpu.sync_copy(acc.at[pl.ds(r, RO)],
                                out.at[p, c, pl.ds(r, RO)])
            if p == 0:
                plsc.subcore_barrier()

    return k


def _tc_pre(N, R, DF, EMB):
    def body(x_ref, win_ref, bin_ref, hA_ref, hB_ref):
        h = jnp.dot(x_ref[...], win_ref[...],
                    preferred_element_type=jnp.float32) + bin_ref[...]
        hA_ref[...] = h[:, :WH]
        hB_ref[...] = jnp.concatenate(
            [h[:, WH:], jnp.zeros((R, 2 * WH - EMB), jnp.float32)], axis=1)

    return pl.pallas_call(
        body,
        grid=(N // R,),
        in_specs=[
            pl.BlockSpec((R, DF), lambda i: (i, 0)),
            pl.BlockSpec((DF, EMB), lambda i: (0, 0)),
            pl.BlockSpec((1, EMB), lambda i: (0, 0)),
        ],
        out_specs=[
            pl.BlockSpec((R, WH), lambda i: (i, 0)),
            pl.BlockSpec((R, WH), lambda i: (i, 0)),
        ],
        out_shape=[
            jax.ShapeDtypeStruct((N, WH), jnp.float32),
            jax.ShapeDtypeStruct((N, WH), jnp.float32),
        ],
    )


def _tc_mlp(N, R, EMB, DE, PROJ, last):
    def body(aggp_ref, ap_ref, we_ref, w1_ref, b1_ref, w2_ref, b2_ref,
             *rest):
        ap = aggp_ref[...]
        agg_a = ap[0, 0] + ap[0, 1]
        agg_b = (ap[1, 0] - ap[0, 0]) + (ap[1, 1] - ap[0, 1])
        agg = jnp.concatenate([agg_a, agg_b[:, :EMB - WH]], axis=1)
        ep = ap_ref[...]
        agg = agg + jnp.dot(ep[0] + ep[1], we_ref[...],
                            preferred_element_type=jnp.float32)
        y = jnp.maximum(jnp.dot(agg, w1_ref[...],
                                preferred_element_type=jnp.float32)
                        + b1_ref[...], 0.0)
        z = jnp.dot(y, w2_ref[...],
                    preferred_element_type=jnp.float32) + b2_ref[...]
        if last:
            wp_ref, bp_ref, out_ref = rest
            o = jnp.dot(z, wp_ref[...],
                        preferred_element_type=jnp.float32) + bp_ref[...]
            n = jnp.maximum(
                jnp.sqrt(jnp.sum(o * o, axis=1, keepdims=True)), 1e-12)
            out_ref[...] = o / n
        else:
            hA_ref, hB_ref = rest
            h = jnp.maximum(z, 0.0)
            hA_ref[...] = h[:, :WH]
            hB_ref[...] = jnp.concatenate(
                [h[:, WH:], jnp.zeros((R, 2 * WH - EMB), jnp.float32)],
                axis=1)

    in_specs = [
        pl.BlockSpec((2, NC, R, WH), lambda i: (0, 0, i, 0)),
        pl.BlockSpec((NC, R, DE), lambda i: (0, i, 0)),
        pl.BlockSpec((DE, EMB), lambda i: (0, 0)),
        pl.BlockSpec((EMB, 2 * EMB), lambda i: (0, 0)),
        pl.BlockSpec((1, 2 * EMB), lambda i: (0, 0)),
        pl.BlockSpec((2 * EMB, EMB), lambda i: (0, 0)),
        pl.BlockSpec((1, EMB), lambda i: (0, 0)),
    ]
    if last:
        in_specs += [
            pl.BlockSpec((EMB, PROJ), lambda i: (0, 0)),
            pl.BlockSpec((1, PROJ), lambda i: (0, 0)),
        ]
        out_specs = pl.BlockSpec((R, PROJ), lambda i: (i, 0))
        out_shape = jax.ShapeDtypeStruct((N, PROJ), jnp.float32)
    else:
        out_specs = [
            pl.BlockSpec((R, WH), lambda i: (i, 0)),
            pl.BlockSpec((R, WH), lambda i: (i, 0)),
        ]
        out_shape = [
            jax.ShapeDtypeStruct((N, WH), jnp.float32),
            jax.ShapeDtypeStruct((N, WH), jnp.float32),
        ]
    return pl.pallas_call(
        body,
        grid=(N // R,),
        in_specs=in_specs,
        out_specs=out_specs,
        out_shape=out_shape,
    )


def kernel(x, edge_index, edge_attr, batch, W_in, b_in, W_edge, W1, b1,
           W2, b2, Wp, bp):
    N, DF = x.shape
    E = edge_index.shape[1]
    DE = edge_attr.shape[1]
    L, _, EMB = W_edge.shape
    PROJ = Wp.shape[1]
    R = 2000
    assert E % (NW * CH) == 0 and N % (NS * 125) == 0 and N % R == 0

    src = edge_index[0]
    dst = edge_index[1]
    src2 = src.reshape(E // CH, CH)
    dst2 = dst.reshape(E // CH, CH)

    ap = _sc_edge_sum(N, E, DE)(edge_attr, dst2)
    hA, hB = _tc_pre(N, R, DF, EMB)(x, W_in, b_in.reshape(1, EMB))

    layer_sc = _sc_layer(N, E)
    for l in range(L):
        parts = layer_sc(hA, hB, src2, dst2)
        args = (parts, ap, W_edge[l], W1[l], b1[l].reshape(1, 2 * EMB),
                W2[l], b2[l].reshape(1, EMB))
        if l < L - 1:
            hA, hB = _tc_mlp(N, R, EMB, DE, PROJ, last=False)(*args)
        else:
            return _tc_mlp(N, R, EMB, DE, PROJ, last=True)(
                *args, Wp, bp.reshape(1, PROJ))


# speedup vs baseline: 8.2401x; 1.0074x over previous
import functools

import jax
import jax.numpy as jnp
from jax import lax
from jax.experimental import pallas as pl
from jax.experimental.pallas import tpu as pltpu
from jax.experimental.pallas import tpu_sc as plsc

NC = 2
NS = 16
NW = NC * NS
CH = 80
WH = 160


def _mesh():
    return plsc.VectorSubcoreMesh(
        core_axis_name="c", subcore_axis_name="s",
        num_cores=NC, num_subcores=NS)


def _zero_vmem(ref, rows, width):
    zv = jnp.zeros((16,), jnp.float32)

    def zi(i, _):
        def zj(j, _):
            ref[i, pl.ds(j * 16, 16)] = zv
            return 0
        return lax.fori_loop(0, width // 16, zj, 0)

    lax.fori_loop(0, rows, zi, 0)


def _sc_edge_sum(N, E, DE):
    EW = E // NW
    NCH = EW // CH
    GB = 25
    NG = NCH // GB
    GE = GB * CH
    RPT = N // NS

    @functools.partial(
        pl.kernel,
        out_type=jax.ShapeDtypeStruct((NC, N, DE), jnp.float32),
        mesh=_mesh(),
        compiler_params=pltpu.CompilerParams(use_tc_tiling_on_sc=False),
        scratch_types=[
            pltpu.VMEM((GE, DE), jnp.float32),
            pltpu.VMEM((GE, DE), jnp.float32),
            pltpu.VMEM((GB, CH), jnp.int32),
            pltpu.VMEM((GB, CH), jnp.int32),
            pltpu.VMEM((RPT, DE), jnp.float32),
            pltpu.VMEM_SHARED((N, DE), jnp.float32),
            pltpu.SemaphoreType.DMA,
            pltpu.SemaphoreType.DMA,
        ],
    )
    def k(ea_hbm, dst_hbm, out, ea0, ea1, d0, d1, zb, acc, isem, ssem):
        c = lax.axis_index("c")
        s = lax.axis_index("s")
        wid = c * NS + s
        _zero_vmem(zb, RPT, DE)
        pltpu.sync_copy(zb, acc.at[pl.ds(s * RPT, RPT)])
        plsc.subcore_barrier()

        eab = (ea0, ea1)
        db = (d0, d1)
        pltpu.sync_copy(ea_hbm.at[pl.ds(wid * EW, GE)], ea0)
        pltpu.sync_copy(dst_hbm.at[pl.ds(wid * NCH, GB)], d0)
        for g in range(NG):
            ea, dst2d = eab[g % 2], db[g % 2]
            if g < NG - 1:
                base = wid * EW + (g + 1) * GE
                pltpu.async_copy(ea_hbm.at[pl.ds(base, GE)],
                                 eab[(g + 1) % 2], isem)
                pltpu.async_copy(dst_hbm.at[pl.ds(wid * NCH + (g + 1) * GB,
                                                  GB)],
                                 db[(g + 1) % 2], isem)

            def fire(j, _):
                pltpu.async_copy(ea.at[pl.ds(j * CH, CH)],
                                 acc.at[dst2d.at[j]], ssem, add=True)
                return 0

            lax.fori_loop(0, GB, fire, 0)

            def drain(j, _):
                pltpu.make_async_copy(ea.at[pl.ds(0, CH)],
                                      acc.at[dst2d.at[0]], ssem).wait()
                return 0

            lax.fori_loop(0, GB, drain, 0)
            if g < NG - 1:
                pltpu.make_async_copy(
                    ea_hbm.at[pl.ds(wid * EW, GE)],
                    eab[(g + 1) % 2], isem).wait()
                pltpu.make_async_copy(
                    dst_hbm.at[pl.ds(wid * NCH, GB)],
                    db[(g + 1) % 2], isem).wait()

        plsc.subcore_barrier()
        pltpu.sync_copy(acc.at[pl.ds(s * RPT, RPT)],
                        out.at[c, pl.ds(s * RPT, RPT)])

    return k


def _sc_layer(N, E):
    EW = E // NW
    NCH = EW // CH
    GB = 25
    NG = NCH // GB
    RPT = N // NS
    RO = 125

    @functools.partial(
        pl.kernel,
        out_type=jax.ShapeDtypeStruct((2, NC, N, WH), jnp.float32),
        mesh=_mesh(),
        compiler_params=pltpu.CompilerParams(use_tc_tiling_on_sc=False),
        scratch_types=[
            pltpu.VMEM((CH, WH), jnp.float32),
            pltpu.VMEM((CH, WH), jnp.float32),
            pltpu.VMEM((GB, CH), jnp.int32),
            pltpu.VMEM((GB, CH), jnp.int32),
            pltpu.VMEM((CH,), jnp.int32),
            pltpu.VMEM((CH,), jnp.int32),
            pltpu.VMEM((CH,), jnp.int32),
            pltpu.VMEM((CH,), jnp.int32),
            pltpu.VMEM_SHARED((N, WH), jnp.float32),
            pltpu.SemaphoreType.DMA,
            pltpu.SemaphoreType.DMA,
            pltpu.SemaphoreType.DMA,
            pltpu.SemaphoreType.DMA,
        ],
    )
    def k(hA, hB, src_hbm, dst_hbm, out, r0, r1, src2d, dst2d,
          s23, s24, d23, d24, acc, g0, g1, isem, osem):
        c = lax.axis_index("c")
        s = lax.axis_index("s")
        wid = c * NS + s

        def fire_idx(grow):
            pltpu.async_copy(src_hbm.at[pl.ds(grow, GB)], src2d, isem)
            pltpu.async_copy(dst_hbm.at[pl.ds(grow, GB)], dst2d, isem)

        def wait_idx(grow):
            pltpu.make_async_copy(
                src_hbm.at[pl.ds(grow, GB)], src2d, isem).wait()
            pltpu.make_async_copy(
                dst_hbm.at[pl.ds(grow, GB)], dst2d, isem).wait()

        def group_head(p, g):
            h = hA if p == 0 else hB
            wait_idx(wid * NCH + g * GB)
            for kk in range(CH // 16):
                sl = pl.ds(kk * 16, 16)
                s23[sl] = src2d[GB - 2, sl]
                s24[sl] = src2d[GB - 1, sl]
                d23[sl] = dst2d[GB - 2, sl]
                d24[sl] = dst2d[GB - 1, sl]
            pltpu.async_copy(h.at[src2d.at[0]], r0, g0)

        fire_idx(wid * NCH)
        _zero_vmem(r1, CH, WH)
        for j in range(RPT // CH):
            pltpu.async_copy(r1, acc.at[pl.ds(s * RPT + j * CH, CH)], osem)
        rem = RPT - (RPT // CH) * CH
        if rem:
            pltpu.async_copy(r1.at[pl.ds(0, rem)],
                             acc.at[pl.ds(s * RPT + RPT - rem, rem)], osem)
        group_head(0, 0)
        for j in range(RPT // CH):
            pltpu.make_async_copy(
                r1, acc.at[pl.ds(s * RPT + j * CH, CH)], osem).wait()
        if rem:
            pltpu.make_async_copy(
                r1.at[pl.ds(0, rem)],
                acc.at[pl.ds(s * RPT + RPT - rem, rem)], osem).wait()
        plsc.subcore_barrier()
        for p in range(2):
            h = hA if p == 0 else hB
            for g in range(NG):
                if g > 0:
                    group_head(p, g)

                def body(t, _):
                    j = 2 * t
                    pltpu.async_copy(h.at[src2d.at[j + 1]], r1, g1)
                    pltpu.make_async_copy(h.at[src2d.at[j]], r0, g0).wait()
                    pltpu.sync_copy(r0, acc.at[dst2d.at[j]], add=True)
                    pltpu.async_copy(h.at[src2d.at[j + 2]], r0, g0)
                    pltpu.make_async_copy(
                        h.at[src2d.at[j + 1]], r1, g1).wait()
                    pltpu.sync_copy(r1, acc.at[dst2d.at[j + 1]], add=True)
                    return 0

                lax.fori_loop(0, (GB - 3) // 2, body, 0)
                pltpu.async_copy(h.at[s23], r1, g1)
                pltpu.make_async_copy(
                    h.at[src2d.at[GB - 3]], r0, g0).wait()
                pltpu.sync_copy(r0, acc.at[dst2d.at[GB - 3]], add=True)
                nxt = p * NG + g + 1
                if nxt < 2 * NG:
                    fire_idx(wid * NCH + (nxt % NG) * GB)
                pltpu.async_copy(h.at[s24], r0, g0)
                pltpu.make_async_copy(h.at[s23], r1, g1).wait()
                pltpu.sync_copy(r1, acc.at[d23], add=True)
                pltpu.make_async_copy(h.at[s24], r0, g0).wait()
                pltpu.sync_copy(r0, acc.at[d24], add=True)

            plsc.subcore_barrier()
            if p == 0:
                for j in range(RPT // RO):
                    r = s * RPT + j * RO
                    pltpu.async_copy(acc.at[pl.ds(r, RO)],
                                     out.at[p, c, pl.ds(r, RO)], osem)
                group_head(1, 0)
                for j in range(RPT // RO):
                    r = s * RPT + j * RO
                    pltpu.make_async_copy(
                        acc.at[pl.ds(r, RO)],
                        out.at[p, c, pl.ds(r, RO)], osem).wait()
                plsc.subcore_barrier()
            else:
                for j in range(RPT // RO):
                    r = s * RPT + j * RO
                    pltpu.sync_copy(acc.at[pl.ds(r, RO)],
                                    out.at[p, c, pl.ds(r, RO)])

    return k


def _tc_pre(N, R, DF, EMB):
    def body(x_ref, win_ref, bin_ref, hA_ref, hB_ref):
        h = jnp.dot(x_ref[...], win_ref[...],
                    preferred_element_type=jnp.float32) + bin_ref[...]
        hA_ref[...] = h[:, :WH]
        hB_ref[...] = jnp.concatenate(
            [h[:, WH:], jnp.zeros((R, 2 * WH - EMB), jnp.float32)], axis=1)

    return pl.pallas_call(
        body,
        grid=(N // R,),
        in_specs=[
            pl.BlockSpec((R, DF), lambda i: (i, 0)),
            pl.BlockSpec((DF, EMB), lambda i: (0, 0)),
            pl.BlockSpec((1, EMB), lambda i: (0, 0)),
        ],
        out_specs=[
            pl.BlockSpec((R, WH), lambda i: (i, 0)),
            pl.BlockSpec((R, WH), lambda i: (i, 0)),
        ],
        out_shape=[
            jax.ShapeDtypeStruct((N, WH), jnp.float32),
            jax.ShapeDtypeStruct((N, WH), jnp.float32),
        ],
    )


def _tc_mlp(N, R, EMB, DE, PROJ, last):
    def body(aggp_ref, ap_ref, we_ref, w1_ref, b1_ref, w2_ref, b2_ref,
             *rest):
        ap = aggp_ref[...]
        agg_a = ap[0, 0] + ap[0, 1]
        agg_b = (ap[1, 0] - ap[0, 0]) + (ap[1, 1] - ap[0, 1])
        agg = jnp.concatenate([agg_a, agg_b[:, :EMB - WH]], axis=1)
        ep = ap_ref[...]
        agg = agg + jnp.dot(ep[0] + ep[1], we_ref[...],
                            preferred_element_type=jnp.float32)
        y = jnp.maximum(jnp.dot(agg, w1_ref[...],
                                preferred_element_type=jnp.float32)
                        + b1_ref[...], 0.0)
        z = jnp.dot(y, w2_ref[...],
                    preferred_element_type=jnp.float32) + b2_ref[...]
        if last:
            wp_ref, bp_ref, out_ref = rest
            o = jnp.dot(z, wp_ref[...],
                        preferred_element_type=jnp.float32) + bp_ref[...]
            n = jnp.maximum(
                jnp.sqrt(jnp.sum(o * o, axis=1, keepdims=True)), 1e-12)
            out_ref[...] = o / n
        else:
            hA_ref, hB_ref = rest
            h = jnp.maximum(z, 0.0)
            hA_ref[...] = h[:, :WH]
            hB_ref[...] = jnp.concatenate(
                [h[:, WH:], jnp.zeros((R, 2 * WH - EMB), jnp.float32)],
                axis=1)

    in_specs = [
        pl.BlockSpec((2, NC, R, WH), lambda i: (0, 0, i, 0)),
        pl.BlockSpec((NC, R, DE), lambda i: (0, i, 0)),
        pl.BlockSpec((DE, EMB), lambda i: (0, 0)),
        pl.BlockSpec((EMB, 2 * EMB), lambda i: (0, 0)),
        pl.BlockSpec((1, 2 * EMB), lambda i: (0, 0)),
        pl.BlockSpec((2 * EMB, EMB), lambda i: (0, 0)),
        pl.BlockSpec((1, EMB), lambda i: (0, 0)),
    ]
    if last:
        in_specs += [
            pl.BlockSpec((EMB, PROJ), lambda i: (0, 0)),
            pl.BlockSpec((1, PROJ), lambda i: (0, 0)),
        ]
        out_specs = pl.BlockSpec((R, PROJ), lambda i: (i, 0))
        out_shape = jax.ShapeDtypeStruct((N, PROJ), jnp.float32)
    else:
        out_specs = [
            pl.BlockSpec((R, WH), lambda i: (i, 0)),
            pl.BlockSpec((R, WH), lambda i: (i, 0)),
        ]
        out_shape = [
            jax.ShapeDtypeStruct((N, WH), jnp.float32),
            jax.ShapeDtypeStruct((N, WH), jnp.float32),
        ]
    return pl.pallas_call(
        body,
        grid=(N // R,),
        in_specs=in_specs,
        out_specs=out_specs,
        out_shape=out_shape,
    )


def kernel(x, edge_index, edge_attr, batch, W_in, b_in, W_edge, W1, b1,
           W2, b2, Wp, bp):
    N, DF = x.shape
    E = edge_index.shape[1]
    DE = edge_attr.shape[1]
    L, _, EMB = W_edge.shape
    PROJ = Wp.shape[1]
    R = 2000
    assert E % (NW * CH) == 0 and N % (NS * 125) == 0 and N % R == 0

    src = edge_index[0]
    dst = edge_index[1]
    src2 = src.reshape(E // CH, CH)
    dst2 = dst.reshape(E // CH, CH)

    ap = _sc_edge_sum(N, E, DE)(edge_attr, dst2)
    hA, hB = _tc_pre(N, R, DF, EMB)(x, W_in, b_in.reshape(1, EMB))

    layer_sc = _sc_layer(N, E)
    for l in range(L):
        parts = layer_sc(hA, hB, src2, dst2)
        args = (parts, ap, W_edge[l], W1[l], b1[l].reshape(1, 2 * EMB),
                W2[l], b2[l].reshape(1, EMB))
        if l < L - 1:
            hA, hB = _tc_mlp(N, R, EMB, DE, PROJ, last=False)(*args)
        else:
            return _tc_mlp(N, R, EMB, DE, PROJ, last=True)(
                *args, Wp, bp.reshape(1, PROJ))
